# speedup vs baseline: 1.0095x; 1.0095x over previous
import functools

import jax
import jax.numpy as jnp
from jax import lax
from jax.experimental import pallas as pl
from jax.experimental.pallas import tpu as pltpu
from jax.experimental.pallas import tpu_sc as plsc

VOCAB = 50257
HIDDEN = 768
MAX_POS = 8192
BATCH = 4
SEQ = 2048

NUM_CORES = 2
NUM_SUBCORES = 16
NUM_WORKERS = NUM_CORES * NUM_SUBCORES
TOTAL = BATCH * SEQ
PER_WORKER = TOTAL // NUM_WORKERS
CHUNK = 64
NCHUNKS = PER_WORKER // CHUNK
LANES = 16
VECS_PER_ROW = HIDDEN // LANES


def _emb_body(ids_hbm, tok_hbm, pos_hbm, out_hbm, idx_v, tok_v, pos_v, sem):
    wid = lax.axis_index("s") * NUM_CORES + lax.axis_index("c")
    base = wid * PER_WORKER
    pos_base = base % SEQ

    for c in range(NCHUNKS):
        row0 = base + c * CHUNK
        prow0 = pos_base + c * CHUNK
        pltpu.sync_copy(ids_hbm.at[pl.ds(row0, CHUNK)], idx_v.at[c])
        pltpu.async_copy(tok_hbm.at[idx_v.at[c]], tok_v, sem).wait()
        pltpu.sync_copy(pos_hbm.at[pl.ds(prow0, CHUNK)], pos_v)

        def add_row(r, _):
            for j in range(VECS_PER_ROW):
                sl = pl.ds(j * LANES, LANES)
                tok_v[r, sl] = tok_v[r, sl] + pos_v[r, sl]
            return 0

        lax.fori_loop(0, CHUNK, add_row, 0)

        pltpu.sync_copy(tok_v, out_hbm.at[pl.ds(row0, CHUNK)])


@jax.jit
def _emb(ids_flat, token_table, pos_table):
    mesh = plsc.VectorSubcoreMesh(core_axis_name="c", subcore_axis_name="s")
    k = functools.partial(
        pl.kernel,
        out_type=jax.ShapeDtypeStruct((TOTAL, HIDDEN), jnp.float32),
        mesh=mesh,
        scratch_types=[
            pltpu.VMEM((NCHUNKS, CHUNK), jnp.int32),
            pltpu.VMEM((CHUNK, HIDDEN), jnp.float32),
            pltpu.VMEM((CHUNK, HIDDEN), jnp.float32),
            pltpu.SemaphoreType.DMA,
        ],
    )(_emb_body)
    return k(ids_flat, token_table, pos_table)


def kernel(input_ids, token_table, pos_table):
    ids_flat = input_ids.reshape(-1).astype(jnp.int32)
    out = _emb(ids_flat, token_table, pos_table)
    return out.reshape(BATCH, SEQ, HIDDEN)


# speedup vs baseline: 1.0732x; 1.0631x over previous
import functools

import jax
import jax.numpy as jnp
from jax import lax
from jax.experimental import pallas as pl
from jax.experimental.pallas import tpu as pltpu
from jax.experimental.pallas import tpu_sc as plsc

VOCAB = 50257
HIDDEN = 768
MAX_POS = 8192
BATCH = 4
SEQ = 2048

NUM_CORES = 2
NUM_SUBCORES = 16
NUM_WORKERS = NUM_CORES * NUM_SUBCORES
TOTAL = BATCH * SEQ
PER_WORKER = TOTAL // NUM_WORKERS
CHUNK = 32
NCHUNKS = PER_WORKER // CHUNK
LANES = 16
VECS_PER_ROW = HIDDEN // LANES
NTOK = 3
NPOS = 2


def _emb_body(ids_hbm, tok_hbm, pos_hbm, out_hbm,
              idx_v, tok_bufs, pos_bufs, gsems, psems, osems):
    wid = lax.axis_index("s") * NUM_CORES + lax.axis_index("c")
    base = wid * PER_WORKER
    pos_base = base % SEQ

    pltpu.sync_copy(ids_hbm.at[pl.ds(base, PER_WORKER)], idx_v)

    gh = [None] * NCHUNKS
    ph = [None] * NCHUNKS
    oh = [None] * NCHUNKS

    def start(c):
        tb = c % NTOK
        pb = c % NPOS
        gh[c] = pltpu.async_copy(
            tok_hbm.at[idx_v.at[pl.ds(c * CHUNK, CHUNK)]],
            tok_bufs.at[tb], gsems.at[tb])
        ph[c] = pltpu.async_copy(
            pos_hbm.at[pl.ds(pos_base + c * CHUNK, CHUNK)],
            pos_bufs.at[pb], psems.at[pb])

    start(0)
    start(1)
    for c in range(NCHUNKS):
        tb = c % NTOK
        pb = c % NPOS
        gh[c].wait()
        ph[c].wait()

        def add_row(r, _):
            for j in range(VECS_PER_ROW):
                sl = pl.ds(j * LANES, LANES)
                plsc.addupdate(tok_bufs.at[tb, r, sl], pos_bufs[pb, r, sl])
            return 0

        lax.fori_loop(0, CHUNK, add_row, 0)

        oh[c] = pltpu.async_copy(
            tok_bufs.at[tb], out_hbm.at[pl.ds(base + c * CHUNK, CHUNK)],
            osems.at[tb])
        nc = c + 2
        if nc < NCHUNKS:
            if nc >= NTOK:
                oh[nc - NTOK].wait()
            start(nc)

    for c in range(NCHUNKS - NTOK, NCHUNKS):
        oh[c].wait()


@jax.jit
def _emb(ids_flat, token_table, pos_table):
    mesh = plsc.VectorSubcoreMesh(core_axis_name="c", subcore_axis_name="s")
    k = functools.partial(
        pl.kernel,
        out_type=jax.ShapeDtypeStruct((TOTAL, HIDDEN), jnp.float32),
        mesh=mesh,
        scratch_types=[
            pltpu.VMEM((PER_WORKER,), jnp.int32),
            pltpu.VMEM((NTOK, CHUNK, HIDDEN), jnp.float32),
            pltpu.VMEM((NPOS, CHUNK, HIDDEN), jnp.float32),
            pltpu.SemaphoreType.DMA((NTOK,)),
            pltpu.SemaphoreType.DMA((NPOS,)),
            pltpu.SemaphoreType.DMA((NTOK,)),
        ],
    )(_emb_body)
    return k(ids_flat, token_table, pos_table)


def kernel(input_ids, token_table, pos_table):
    ids_flat = input_ids.reshape(-1).astype(jnp.int32)
    out = _emb(ids_flat, token_table, pos_table)
    return out.reshape(BATCH, SEQ, HIDDEN)


# speedup vs baseline: 1.1448x; 1.0667x over previous
import functools

import jax
import jax.numpy as jnp
from jax import lax
from jax.experimental import pallas as pl
from jax.experimental.pallas import tpu as pltpu
from jax.experimental.pallas import tpu_sc as plsc

VOCAB = 50257
HIDDEN = 768
MAX_POS = 8192
BATCH = 4
SEQ = 2048

NUM_CORES = 2
NUM_SUBCORES = 16
NUM_WORKERS = NUM_CORES * NUM_SUBCORES
TOTAL = BATCH * SEQ
PER_WORKER = TOTAL // NUM_WORKERS
CHUNK = 32
NCHUNKS = PER_WORKER // CHUNK
LANES = 16
VECS_PER_ROW = HIDDEN // LANES
NTOK = 2
NPOS = 3


def _emb_body(ids_hbm, tok_hbm, pos_hbm, out_hbm,
              idx_v, tok_bufs, pos_bufs, gsems, psems, osems):
    wid = lax.axis_index("s") * NUM_CORES + lax.axis_index("c")
    base = wid * PER_WORKER
    pos_base = base % SEQ

    pltpu.sync_copy(ids_hbm.at[pl.ds(base, PER_WORKER)], idx_v)

    gh = [None] * NCHUNKS
    ph = [None] * NCHUNKS
    oh = [None] * NCHUNKS

    def start_gather(c):
        tb = c % NTOK
        gh[c] = pltpu.async_copy(
            tok_hbm.at[idx_v.at[pl.ds(c * CHUNK, CHUNK)]],
            tok_bufs.at[tb], gsems.at[tb])

    def start_pos(c):
        pb = c % NPOS
        ph[c] = pltpu.async_copy(
            pos_hbm.at[pl.ds(pos_base + c * CHUNK, CHUNK)],
            pos_bufs.at[pb], psems.at[pb])

    start_gather(0)
    start_pos(0)
    start_gather(1)
    start_pos(1)
    for c in range(NCHUNKS):
        tb = c % NTOK
        pb = c % NPOS
        gh[c].wait()
        ph[c].wait()

        @plsc.parallel_loop(0, CHUNK, unroll=2)
        def add_row(r):
            for j in range(VECS_PER_ROW):
                sl = pl.ds(j * LANES, LANES)
                plsc.addupdate(pos_bufs.at[pb, r, sl], tok_bufs[tb, r, sl])

        oh[c] = pltpu.async_copy(
            pos_bufs.at[pb], out_hbm.at[pl.ds(base + c * CHUNK, CHUNK)],
            osems.at[pb])
        nc = c + 2
        if nc < NCHUNKS:
            start_gather(nc)
            if nc >= NPOS:
                oh[nc - NPOS].wait()
            start_pos(nc)

    for c in range(NCHUNKS - NPOS, NCHUNKS):
        oh[c].wait()


@jax.jit
def _emb(ids_flat, token_table, pos_table):
    mesh = plsc.VectorSubcoreMesh(core_axis_name="c", subcore_axis_name="s")
    k = functools.partial(
        pl.kernel,
        out_type=jax.ShapeDtypeStruct((TOTAL, HIDDEN), jnp.float32),
        mesh=mesh,
        scratch_types=[
            pltpu.VMEM((PER_WORKER,), jnp.int32),
            pltpu.VMEM((NTOK, CHUNK, HIDDEN), jnp.float32),
            pltpu.VMEM((NPOS, CHUNK, HIDDEN), jnp.float32),
            pltpu.SemaphoreType.DMA((NTOK,)),
            pltpu.SemaphoreType.DMA((NPOS,)),
            pltpu.SemaphoreType.DMA((NPOS,)),
        ],
    )(_emb_body)
    return k(ids_flat, token_table, pos_table)


def kernel(input_ids, token_table, pos_table):
    ids_flat = input_ids.reshape(-1).astype(jnp.int32)
    out = _emb(ids_flat, token_table, pos_table)
    return out.reshape(BATCH, SEQ, HIDDEN)
